# Initial kernel scaffold; baseline (speedup 1.0000x reference)
import jax
import jax.numpy as jnp
from jax.experimental import pallas as pl


def kernel(x, edge_index, W1, b1, W2, b2):
    raise NotImplementedError("write your pallas kernel here")


# speedup vs baseline: 12.7827x; 12.7827x over previous
import functools

import jax
import jax.numpy as jnp
from jax import lax
from jax.experimental import pallas as pl
from jax.experimental.pallas import tpu as pltpu
from jax.experimental.pallas import tpu_sc as plsc

NC = 2
NS = 16
NW = NC * NS


def _deg_kernel(n_pad, e, chunk):
  e_per_w = e // NW
  nch = e_per_w // chunk
  rows_per_tec = n_pad // NS

  mesh = plsc.VectorSubcoreMesh(
      core_axis_name="c", subcore_axis_name="s",
      num_cores=NC, num_subcores=NS)

  @functools.partial(
      pl.kernel,
      out_type=jax.ShapeDtypeStruct((NC * n_pad,), jnp.float32),
      mesh=mesh,
      scratch_types=[
          pltpu.VMEM((chunk,), jnp.int32),
          pltpu.VMEM((chunk,), jnp.float32),
          pltpu.VMEM((rows_per_tec,), jnp.float32),
          pltpu.VMEM_SHARED((n_pad,), jnp.float32),
      ],
  )
  def k(dst_hbm, out_hbm, didx, ones, fbuf, acc):
    core = lax.axis_index("c")
    sub = lax.axis_index("s")
    wid = core * NS + sub

    def fill(i, _):
      ones[pl.ds(i * 16, 16)] = jnp.full((16,), 1.0, jnp.float32)
      return 0
    lax.fori_loop(0, chunk // 16, fill, 0)

    def zero(i, _):
      fbuf[pl.ds(i * 16, 16)] = jnp.zeros((16,), jnp.float32)
      return 0
    lax.fori_loop(0, rows_per_tec // 16, zero, 0)
    pltpu.sync_copy(fbuf, acc.at[pl.ds(sub * rows_per_tec, rows_per_tec)])
    plsc.subcore_barrier()

    def body(i, _):
      base = wid * e_per_w + i * chunk
      pltpu.sync_copy(dst_hbm.at[pl.ds(base, chunk)], didx)
      pltpu.sync_copy(ones, acc.at[didx], add=True)
      return 0
    lax.fori_loop(0, nch, body, 0)
    plsc.subcore_barrier()

    pltpu.sync_copy(acc.at[pl.ds(sub * rows_per_tec, rows_per_tec)], fbuf)
    pltpu.sync_copy(
        fbuf,
        out_hbm.at[pl.ds(core * n_pad + sub * rows_per_tec, rows_per_tec)])

  return k


def _edge_agg_kernel(n_pad, d, e, chunk, fch):
  e_per_w = e // NW
  nch = e_per_w // chunk
  rows_per_tec = n_pad // NS
  nf = rows_per_tec // fch

  mesh = plsc.VectorSubcoreMesh(
      core_axis_name="c", subcore_axis_name="s",
      num_cores=NC, num_subcores=NS)

  @functools.partial(
      pl.kernel,
      out_type=jax.ShapeDtypeStruct((NC, n_pad, d), jnp.float32),
      mesh=mesh,
      scratch_types=[
          pltpu.VMEM((chunk,), jnp.int32),
          pltpu.VMEM((chunk,), jnp.int32),
          pltpu.VMEM((chunk, d), jnp.float32),
          pltpu.VMEM((fch, d), jnp.float32),
          pltpu.VMEM_SHARED((n_pad, d), jnp.float32),
          pltpu.SemaphoreType.DMA,
      ],
  )
  def k(g_hbm, src_hbm, dst_hbm, out_hbm, sidx, didx, rows, fbuf, acc, sem):
    core = lax.axis_index("c")
    sub = lax.axis_index("s")
    wid = core * NS + sub

    def zero(i, _):
      r = i // (d // 16)
      c = (i % (d // 16)) * 16
      fbuf[r, pl.ds(c, 16)] = jnp.zeros((16,), jnp.float32)
      return 0
    lax.fori_loop(0, fch * (d // 16), zero, 0)
    for f in range(nf):
      r0 = sub * rows_per_tec + f * fch
      pltpu.sync_copy(fbuf, acc.at[pl.ds(r0, fch)])
    plsc.subcore_barrier()

    def body(i, _):
      base = wid * e_per_w + i * chunk
      pltpu.sync_copy(src_hbm.at[pl.ds(base, chunk)], sidx)
      pltpu.sync_copy(dst_hbm.at[pl.ds(base, chunk)], didx)
      pltpu.async_copy(g_hbm.at[sidx], rows, sem).wait()
      pltpu.sync_copy(rows, acc.at[didx], add=True)
      return 0
    lax.fori_loop(0, nch, body, 0)
    plsc.subcore_barrier()

    for f in range(nf):
      r0 = sub * rows_per_tec + f * fch
      pltpu.sync_copy(acc.at[pl.ds(r0, fch)], fbuf)
      pltpu.sync_copy(fbuf, out_hbm.at[core, pl.ds(r0, fch)])

  return k


def _dis_from_deg(deg_ref):
  deg = deg_ref[:, 0:1] + deg_ref[:, 1:2] + 1.0
  return lax.rsqrt(jnp.maximum(deg, 1.0))


def _tc1_body(deg_ref, x_ref, w1_ref, g1_ref):
  dis = _dis_from_deg(deg_ref)
  h = jnp.dot(x_ref[...], w1_ref[...], preferred_element_type=jnp.float32,
              precision=lax.Precision.HIGHEST)
  g1_ref[...] = h * dis


def _tc2_body(acc_ref, g1_ref, deg_ref, b1_ref, w2_ref, g2_ref):
  dis = _dis_from_deg(deg_ref)
  s = acc_ref[0] + acc_ref[1] + g1_ref[...]
  out1 = jnp.maximum(s * dis + b1_ref[...], 0.0)
  h2 = jnp.dot(out1, w2_ref[...], preferred_element_type=jnp.float32,
               precision=lax.Precision.HIGHEST)
  g2_ref[...] = h2 * dis


def _tc3_body(acc_ref, g2_ref, deg_ref, b2_ref, o_ref):
  dis = _dis_from_deg(deg_ref)
  s = acc_ref[0] + acc_ref[1] + g2_ref[...]
  o = s * dis + b2_ref[...]
  m = jnp.max(o, axis=1, keepdims=True)
  z = o - m
  lse = jnp.log(jnp.sum(jnp.exp(z), axis=1, keepdims=True))
  o_ref[...] = z - lse


@jax.jit
def kernel(x, edge_index, W1, b1, W2, b2):
  n, d_in = x.shape
  e = edge_index.shape[1]
  d_hid = W1.shape[1]
  d_out = W2.shape[1]

  src = edge_index[0]
  dst = edge_index[1]

  dh = 128
  W1p = jnp.pad(W1, ((0, 0), (0, dh - d_hid)))
  b1p = jnp.pad(b1, (0, dh - d_hid))
  W2p = jnp.pad(W2, ((0, dh - d_hid), (0, 0)))

  n_pad = ((n + (128 * NS) - 1) // (128 * NS)) * (128 * NS)
  chunk = 80
  fch = 128

  deg2 = _deg_kernel(n_pad, e, chunk)(dst)
  deg2 = deg2.reshape(NC, n_pad)[:, :n].T

  bm = 1000
  grid = n // bm

  deg_spec = pl.BlockSpec((bm, NC), lambda i: (i, 0))
  acc_spec_h = pl.BlockSpec((NC, bm, dh), lambda i: (0, i, 0))
  acc_spec_o = pl.BlockSpec((NC, bm, d_out), lambda i: (0, i, 0))

  g1 = pl.pallas_call(
      _tc1_body,
      grid=(grid,),
      in_specs=[
          deg_spec,
          pl.BlockSpec((bm, d_in), lambda i: (i, 0)),
          pl.BlockSpec((d_in, dh), lambda i: (0, 0)),
      ],
      out_specs=pl.BlockSpec((bm, dh), lambda i: (i, 0)),
      out_shape=jax.ShapeDtypeStruct((n, dh), jnp.float32),
  )(deg2, x, W1p)

  acc1 = _edge_agg_kernel(n_pad, dh, e, chunk, fch)(g1, src, dst)[:, :n]

  g2 = pl.pallas_call(
      _tc2_body,
      grid=(grid,),
      in_specs=[
          acc_spec_h,
          pl.BlockSpec((bm, dh), lambda i: (i, 0)),
          deg_spec,
          pl.BlockSpec((1, dh), lambda i: (0, 0)),
          pl.BlockSpec((dh, d_out), lambda i: (0, 0)),
      ],
      out_specs=pl.BlockSpec((bm, d_out), lambda i: (i, 0)),
      out_shape=jax.ShapeDtypeStruct((n, d_out), jnp.float32),
  )(acc1, g1, deg2, b1p.reshape(1, dh), W2p)

  acc2 = _edge_agg_kernel(n_pad, d_out, e, chunk, fch)(g2, src, dst)[:, :n]

  out = pl.pallas_call(
      _tc3_body,
      grid=(grid,),
      in_specs=[
          acc_spec_o,
          pl.BlockSpec((bm, d_out), lambda i: (i, 0)),
          deg_spec,
          pl.BlockSpec((1, d_out), lambda i: (0, 0)),
      ],
      out_specs=pl.BlockSpec((bm, d_out), lambda i: (i, 0)),
      out_shape=jax.ShapeDtypeStruct((n, d_out), jnp.float32),
  )(acc2, g2, deg2, b2.reshape(1, d_out))

  return out


# speedup vs baseline: 19.9574x; 1.5613x over previous
import functools

import jax
import jax.numpy as jnp
from jax import lax
from jax.experimental import pallas as pl
from jax.experimental.pallas import tpu as pltpu
from jax.experimental.pallas import tpu_sc as plsc

NC = 2
NS = 16
NW = NC * NS


_D = 5


def _deg_kernel(n_pad, e, chunk):
  e_per_w = e // NW
  nch = e_per_w // chunk
  rows_per_tec = n_pad // NS

  mesh = plsc.VectorSubcoreMesh(
      core_axis_name="c", subcore_axis_name="s",
      num_cores=NC, num_subcores=NS)

  @functools.partial(
      pl.kernel,
      out_type=jax.ShapeDtypeStruct((NC * n_pad,), jnp.float32),
      mesh=mesh,
      scratch_types=(
          [pltpu.VMEM((chunk,), jnp.int32)] * _D +
          [pltpu.VMEM((((chunk + 15) // 16) * 16,), jnp.float32)] +
          [pltpu.VMEM((rows_per_tec,), jnp.float32)] +
          [pltpu.VMEM_SHARED((n_pad,), jnp.float32)] +
          [pltpu.SemaphoreType.DMA] * (2 * _D)
      ),
  )
  def k(dst_hbm, out_hbm, *scr):
    didx = scr[:_D]
    ones, fbuf, acc = scr[_D], scr[_D + 1], scr[_D + 2]
    sems = scr[_D + 3:]
    isem = sems[:_D]
    ssem = sems[_D:]
    core = lax.axis_index("c")
    sub = lax.axis_index("s")
    wid = core * NS + sub
    estart = wid * e_per_w

    def fill(i, _):
      ones[pl.ds(i * 16, 16)] = jnp.full((16,), 1.0, jnp.float32)
      return 0
    lax.fori_loop(0, (chunk + 15) // 16, fill, 0)

    def zero(i, _):
      fbuf[pl.ds(i * 16, 16)] = jnp.zeros((16,), jnp.float32)
      return 0
    lax.fori_loop(0, rows_per_tec // 16, zero, 0)
    pltpu.sync_copy(fbuf, acc.at[pl.ds(sub * rows_per_tec, rows_per_tec)])
    plsc.subcore_barrier()

    def idx_load(i, b):
      return pltpu.make_async_copy(
          dst_hbm.at[pl.ds(estart + i * chunk, chunk)], didx[b], isem[b])

    for b in range(_D):
      idx_load(b, b).start()

    def group(g, _):
      for b in range(_D):
        i = g * _D + b
        idx_load(i, b).wait()
        sc = pltpu.make_async_copy(
            ones.at[pl.ds(0, chunk)], acc.at[didx[b]], ssem[b])
        sc.start(add=True)
        sc.wait()

        @pl.when(i + _D < nch)
        def _():
          idx_load(i + _D, b).start()
      return 0
    lax.fori_loop(0, nch // _D, group, 0)
    plsc.subcore_barrier()

    pltpu.sync_copy(acc.at[pl.ds(sub * rows_per_tec, rows_per_tec)], fbuf)
    pltpu.sync_copy(
        fbuf,
        out_hbm.at[pl.ds(core * n_pad + sub * rows_per_tec, rows_per_tec)])

  return k


def _edge_agg_kernel(n_pad, d, e, chunk, fch):
  e_per_w = e // NW
  nch = e_per_w // chunk
  rows_per_tec = n_pad // NS
  nf = rows_per_tec // fch

  mesh = plsc.VectorSubcoreMesh(
      core_axis_name="c", subcore_axis_name="s",
      num_cores=NC, num_subcores=NS)

  @functools.partial(
      pl.kernel,
      out_type=jax.ShapeDtypeStruct((NC, n_pad, d), jnp.float32),
      mesh=mesh,
      scratch_types=(
          [pltpu.VMEM((chunk,), jnp.int32)] * _D +
          [pltpu.VMEM((chunk,), jnp.int32)] * _D +
          [pltpu.VMEM((chunk, d), jnp.float32)] * _D +
          [pltpu.VMEM((fch, d), jnp.float32)] +
          [pltpu.VMEM_SHARED((n_pad, d), jnp.float32)] +
          [pltpu.SemaphoreType.DMA] * (4 * _D)
      ),
  )
  def k(g_hbm, src_hbm, dst_hbm, out_hbm, *scr):
    sidx = scr[:_D]
    didx = scr[_D:2 * _D]
    rows = scr[2 * _D:3 * _D]
    fbuf, acc = scr[3 * _D], scr[3 * _D + 1]
    sems = scr[3 * _D + 2:]
    isem_s = sems[:_D]
    isem_d = sems[_D:2 * _D]
    gsem = sems[2 * _D:3 * _D]
    ssem = sems[3 * _D:]
    core = lax.axis_index("c")
    sub = lax.axis_index("s")
    wid = core * NS + sub
    estart = wid * e_per_w

    def zero(i, _):
      r = i // (d // 16)
      c = (i % (d // 16)) * 16
      fbuf[r, pl.ds(c, 16)] = jnp.zeros((16,), jnp.float32)
      return 0
    lax.fori_loop(0, fch * (d // 16), zero, 0)
    for f in range(nf):
      r0 = sub * rows_per_tec + f * fch
      pltpu.sync_copy(fbuf, acc.at[pl.ds(r0, fch)])
    plsc.subcore_barrier()

    def sidx_load(i, b):
      return pltpu.make_async_copy(
          src_hbm.at[pl.ds(estart + i * chunk, chunk)], sidx[b], isem_s[b])

    def didx_load(i, b):
      return pltpu.make_async_copy(
          dst_hbm.at[pl.ds(estart + i * chunk, chunk)], didx[b], isem_d[b])

    def gather(b):
      return pltpu.make_async_copy(g_hbm.at[sidx[b]], rows[b], gsem[b])

    for b in range(_D):
      sidx_load(b, b).start()
      didx_load(b, b).start()
    for b in range(_D - 1):
      sidx_load(b, b).wait()
      gather(b).start()

    def group(g, _):
      for b in range(_D):
        i = g * _D + b
        jb = (b + _D - 1) % _D

        @pl.when(i + _D - 1 < nch)
        def _():
          sidx_load(i + _D - 1, jb).wait()
          gather(jb).start()

        gather(b).wait()
        didx_load(i, b).wait()
        sc = pltpu.make_async_copy(rows[b], acc.at[didx[b]], ssem[b])
        sc.start(add=True)
        sc.wait()

        @pl.when(i + _D < nch)
        def _():
          sidx_load(i + _D, b).start()
          didx_load(i + _D, b).start()
      return 0
    lax.fori_loop(0, nch // _D, group, 0)
    plsc.subcore_barrier()

    for f in range(nf):
      r0 = sub * rows_per_tec + f * fch
      pltpu.sync_copy(acc.at[pl.ds(r0, fch)], fbuf)
      pltpu.sync_copy(fbuf, out_hbm.at[core, pl.ds(r0, fch)])

  return k


def _dis_from_deg(deg_ref):
  deg = deg_ref[:, 0:1] + deg_ref[:, 1:2] + 1.0
  return lax.rsqrt(jnp.maximum(deg, 1.0))


def _tc1_body(deg_ref, x_ref, w1_ref, g1_ref):
  dis = _dis_from_deg(deg_ref)
  h = jnp.dot(x_ref[...], w1_ref[...], preferred_element_type=jnp.float32,
              precision=lax.Precision.HIGHEST)
  g1_ref[...] = h * dis


def _tc2_body(acc_ref, g1_ref, deg_ref, b1_ref, w2_ref, g2_ref):
  dis = _dis_from_deg(deg_ref)
  s = acc_ref[0] + acc_ref[1] + g1_ref[...]
  out1 = jnp.maximum(s * dis + b1_ref[...], 0.0)
  h2 = jnp.dot(out1, w2_ref[...], preferred_element_type=jnp.float32,
               precision=lax.Precision.HIGHEST)
  g2_ref[...] = h2 * dis


def _tc3_body(acc_ref, g2_ref, deg_ref, b2_ref, o_ref):
  dis = _dis_from_deg(deg_ref)
  s = acc_ref[0] + acc_ref[1] + g2_ref[...]
  o = s * dis + b2_ref[...]
  m = jnp.max(o, axis=1, keepdims=True)
  z = o - m
  lse = jnp.log(jnp.sum(jnp.exp(z), axis=1, keepdims=True))
  o_ref[...] = z - lse


@jax.jit
def kernel(x, edge_index, W1, b1, W2, b2):
  n, d_in = x.shape
  e = edge_index.shape[1]
  d_hid = W1.shape[1]
  d_out = W2.shape[1]

  src = edge_index[0]
  dst = edge_index[1]

  dh = 128
  W1p = jnp.pad(W1, ((0, 0), (0, dh - d_hid)))
  b1p = jnp.pad(b1, (0, dh - d_hid))
  W2p = jnp.pad(W2, ((0, dh - d_hid), (0, 0)))

  n_pad = ((n + (128 * NS) - 1) // (128 * NS)) * (128 * NS)
  chunk = 40
  fch = 128

  deg2 = _deg_kernel(n_pad, e, chunk)(dst)
  deg2 = deg2.reshape(NC, n_pad)[:, :n].T

  bm = 1000
  grid = n // bm

  deg_spec = pl.BlockSpec((bm, NC), lambda i: (i, 0))
  acc_spec_h = pl.BlockSpec((NC, bm, dh), lambda i: (0, i, 0))
  acc_spec_o = pl.BlockSpec((NC, bm, d_out), lambda i: (0, i, 0))

  g1 = pl.pallas_call(
      _tc1_body,
      grid=(grid,),
      in_specs=[
          deg_spec,
          pl.BlockSpec((bm, d_in), lambda i: (i, 0)),
          pl.BlockSpec((d_in, dh), lambda i: (0, 0)),
      ],
      out_specs=pl.BlockSpec((bm, dh), lambda i: (i, 0)),
      out_shape=jax.ShapeDtypeStruct((n, dh), jnp.float32),
  )(deg2, x, W1p)

  edge_agg = _edge_agg_kernel(n_pad, dh, e, chunk, fch)
  acc1 = edge_agg(g1, src, dst)[:, :n]

  g2 = pl.pallas_call(
      _tc2_body,
      grid=(grid,),
      in_specs=[
          acc_spec_h,
          pl.BlockSpec((bm, dh), lambda i: (i, 0)),
          deg_spec,
          pl.BlockSpec((1, dh), lambda i: (0, 0)),
          pl.BlockSpec((dh, d_out), lambda i: (0, 0)),
      ],
      out_specs=pl.BlockSpec((bm, d_out), lambda i: (i, 0)),
      out_shape=jax.ShapeDtypeStruct((n, d_out), jnp.float32),
  )(acc1, g1, deg2, b1p.reshape(1, dh), W2p)

  acc2 = edge_agg(g2, src, dst)[:, :n]

  out = pl.pallas_call(
      _tc3_body,
      grid=(grid,),
      in_specs=[
          acc_spec_o,
          pl.BlockSpec((bm, d_out), lambda i: (i, 0)),
          deg_spec,
          pl.BlockSpec((1, d_out), lambda i: (0, 0)),
      ],
      out_specs=pl.BlockSpec((bm, d_out), lambda i: (i, 0)),
      out_shape=jax.ShapeDtypeStruct((n, d_out), jnp.float32),
  )(acc2, g2, deg2, b2.reshape(1, d_out))

  return out


# speedup vs baseline: 32.1880x; 1.6128x over previous
import functools

import jax
import jax.numpy as jnp
from jax import lax
from jax.experimental import pallas as pl
from jax.experimental.pallas import tpu as pltpu
from jax.experimental.pallas import tpu_sc as plsc

NC = 2
NS = 16
NW = NC * NS


_D = 5


def _deg_kernel(n_pad, e, chunk):
  e_per_w = e // NW
  nch = e_per_w // chunk
  rows_per_tec = n_pad // NS

  mesh = plsc.VectorSubcoreMesh(
      core_axis_name="c", subcore_axis_name="s",
      num_cores=NC, num_subcores=NS)

  @functools.partial(
      pl.kernel,
      out_type=jax.ShapeDtypeStruct((NC * n_pad,), jnp.float32),
      mesh=mesh,
      scratch_types=(
          [pltpu.VMEM((chunk,), jnp.int32)] * _D +
          [pltpu.VMEM((((chunk + 15) // 16) * 16,), jnp.float32)] +
          [pltpu.VMEM((rows_per_tec,), jnp.float32)] +
          [pltpu.VMEM_SHARED((n_pad,), jnp.float32)] +
          [pltpu.SemaphoreType.DMA] * (2 * _D)
      ),
  )
  def k(dst_hbm, out_hbm, *scr):
    didx = scr[:_D]
    ones, fbuf, acc = scr[_D], scr[_D + 1], scr[_D + 2]
    sems = scr[_D + 3:]
    isem = sems[:_D]
    ssem = sems[_D:]
    core = lax.axis_index("c")
    sub = lax.axis_index("s")
    wid = core * NS + sub
    estart = wid * e_per_w

    def fill(i, _):
      ones[pl.ds(i * 16, 16)] = jnp.full((16,), 1.0, jnp.float32)
      return 0
    lax.fori_loop(0, (chunk + 15) // 16, fill, 0)

    def zero(i, _):
      fbuf[pl.ds(i * 16, 16)] = jnp.zeros((16,), jnp.float32)
      return 0
    lax.fori_loop(0, rows_per_tec // 16, zero, 0)
    pltpu.sync_copy(fbuf, acc.at[pl.ds(sub * rows_per_tec, rows_per_tec)])
    plsc.subcore_barrier()

    def idx_load(i, b):
      return pltpu.make_async_copy(
          dst_hbm.at[pl.ds(estart + i * chunk, chunk)], didx[b], isem[b])

    for b in range(_D):
      idx_load(b, b).start()

    def group(g, _):
      for b in range(_D):
        i = g * _D + b
        idx_load(i, b).wait()
        sc = pltpu.make_async_copy(
            ones.at[pl.ds(0, chunk)], acc.at[didx[b]], ssem[b])
        sc.start(add=True)
        sc.wait()

        @pl.when(i + _D < nch)
        def _():
          idx_load(i + _D, b).start()
      return 0
    lax.fori_loop(0, nch // _D, group, 0)
    plsc.subcore_barrier()

    pltpu.sync_copy(acc.at[pl.ds(sub * rows_per_tec, rows_per_tec)], fbuf)
    pltpu.sync_copy(
        fbuf,
        out_hbm.at[pl.ds(core * n_pad + sub * rows_per_tec, rows_per_tec)])

  return k


def _edge_agg_kernel(n_pad, d, e, chunk, fch):
  e_per_w = e // NW
  nch = e_per_w // chunk
  rows_per_tec = n_pad // NS
  nf = rows_per_tec // fch

  mesh = plsc.VectorSubcoreMesh(
      core_axis_name="c", subcore_axis_name="s",
      num_cores=NC, num_subcores=NS)

  @functools.partial(
      pl.kernel,
      out_type=jax.ShapeDtypeStruct((NC, n_pad, d), jnp.float32),
      mesh=mesh,
      scratch_types=(
          [pltpu.VMEM((chunk,), jnp.int32)] * (2 * _D) +
          [pltpu.VMEM((chunk,), jnp.int32)] * (2 * _D) +
          [pltpu.VMEM((chunk, d), jnp.float32)] * _D +
          [pltpu.VMEM((fch, d), jnp.float32)] +
          [pltpu.VMEM_SHARED((n_pad, d), jnp.float32)] +
          [pltpu.SemaphoreType.DMA] * (6 * _D)
      ),
  )
  def k(g_hbm, src_hbm, dst_hbm, out_hbm, *scr):
    sidx = scr[:2 * _D]
    didx = scr[2 * _D:4 * _D]
    rows = scr[4 * _D:5 * _D]
    fbuf, acc = scr[5 * _D], scr[5 * _D + 1]
    sems = scr[5 * _D + 2:]
    isem_s = sems[:2 * _D]
    isem_d = sems[2 * _D:4 * _D]
    gsem = sems[4 * _D:5 * _D]
    ssem = sems[5 * _D:]
    core = lax.axis_index("c")
    sub = lax.axis_index("s")
    wid = core * NS + sub
    estart = wid * e_per_w

    def zero(i, _):
      r = i // (d // 16)
      c = (i % (d // 16)) * 16
      fbuf[r, pl.ds(c, 16)] = jnp.zeros((16,), jnp.float32)
      return 0
    lax.fori_loop(0, fch * (d // 16), zero, 0)
    for f in range(nf):
      r0 = sub * rows_per_tec + f * fch
      pltpu.sync_copy(fbuf, acc.at[pl.ds(r0, fch)])
    plsc.subcore_barrier()

    def sidx_load(i, c):
      return pltpu.make_async_copy(
          src_hbm.at[pl.ds(estart + i * chunk, chunk)], sidx[c], isem_s[c])

    def didx_load(i, c):
      return pltpu.make_async_copy(
          dst_hbm.at[pl.ds(estart + i * chunk, chunk)], didx[c], isem_d[c])

    def gather(b, c):
      return pltpu.make_async_copy(g_hbm.at[sidx[c]], rows[b], gsem[b])

    def scatter(b, c):
      return pltpu.make_async_copy(rows[b], acc.at[didx[c]], ssem[b])

    for c in range(2 * _D):
      sidx_load(c, c).start()
      didx_load(c, c).start()
    for b in range(_D - 1):
      sidx_load(b, b).wait()
      gather(b, b).start()

    def group(g, _):
      for h in range(2):
        for b in range(_D):
          i = g * (2 * _D) + h * _D + b
          c = h * _D + b
          jb = (b + _D - 1) % _D
          jc = (c + _D - 1) % (2 * _D)
          pc = (c + 2 * _D - 1) % (2 * _D)

          @pl.when((i + _D - 1 < nch) & (i >= 1))
          def _():
            scatter(jb, pc).wait()

          @pl.when(i + _D - 1 < nch)
          def _():
            sidx_load(i + _D - 1, jc).wait()
            gather(jb, jc).start()

          gather(b, c).wait()
          didx_load(i, c).wait()
          scatter(b, c).start(add=True)

          @pl.when((i >= 1) & (i + 2 * _D - 1 < nch))
          def _():
            sidx_load(i + 2 * _D - 1, pc).start()
            didx_load(i + 2 * _D - 1, pc).start()
      return 0
    lax.fori_loop(0, nch // (2 * _D), group, 0)

    for b in range(_D):
      i = nch - _D + b
      scatter(i % _D, i % (2 * _D)).wait()
    plsc.subcore_barrier()

    for f in range(nf):
      r0 = sub * rows_per_tec + f * fch
      pltpu.sync_copy(acc.at[pl.ds(r0, fch)], fbuf)
      pltpu.sync_copy(fbuf, out_hbm.at[core, pl.ds(r0, fch)])

  return k


def _dis_from_deg(deg_ref):
  deg = deg_ref[:, 0:1] + deg_ref[:, 1:2] + 1.0
  return lax.rsqrt(jnp.maximum(deg, 1.0))


def _tc0_body(x_ref, w1_ref, h1_ref):
  h1_ref[...] = jnp.dot(x_ref[...], w1_ref[...],
                        preferred_element_type=jnp.float32,
                        precision=lax.Precision.HIGHEST)


def _tc1_body(deg_ref, h1_ref, g1_ref):
  dis = _dis_from_deg(deg_ref)
  g1_ref[...] = h1_ref[...] * dis


def _tc2_body(acc_ref, g1_ref, deg_ref, b1_ref, w2_ref, g2_ref):
  dis = _dis_from_deg(deg_ref)
  s = acc_ref[0] + acc_ref[1] + g1_ref[...]
  out1 = jnp.maximum(s * dis + b1_ref[...], 0.0)
  h2 = jnp.dot(out1, w2_ref[...], preferred_element_type=jnp.float32,
               precision=lax.Precision.HIGHEST)
  g2_ref[...] = h2 * dis


def _tc3_body(acc_ref, g2_ref, deg_ref, b2_ref, o_ref):
  dis = _dis_from_deg(deg_ref)
  s = acc_ref[0] + acc_ref[1] + g2_ref[...]
  o = s * dis + b2_ref[...]
  m = jnp.max(o, axis=1, keepdims=True)
  z = o - m
  lse = jnp.log(jnp.sum(jnp.exp(z), axis=1, keepdims=True))
  o_ref[...] = z - lse


@jax.jit
def kernel(x, edge_index, W1, b1, W2, b2):
  n, d_in = x.shape
  e = edge_index.shape[1]
  d_hid = W1.shape[1]
  d_out = W2.shape[1]

  src = edge_index[0]
  dst = edge_index[1]

  dh = 128
  W1p = jnp.pad(W1, ((0, 0), (0, dh - d_hid)))
  b1p = jnp.pad(b1, (0, dh - d_hid))
  W2p = jnp.pad(W2, ((0, dh - d_hid), (0, 0)))

  n_pad = ((n + (128 * NS) - 1) // (128 * NS)) * (128 * NS)
  chunk = 40
  fch = 128

  deg2 = _deg_kernel(n_pad, e, chunk)(dst)
  deg2 = deg2.reshape(NC, n_pad)[:, :n].T

  bm = 1000
  grid = n // bm

  deg_spec = pl.BlockSpec((bm, NC), lambda i: (i, 0))
  acc_spec_h = pl.BlockSpec((NC, bm, dh), lambda i: (0, i, 0))
  acc_spec_o = pl.BlockSpec((NC, bm, d_out), lambda i: (0, i, 0))

  h1 = pl.pallas_call(
      _tc0_body,
      grid=(grid,),
      in_specs=[
          pl.BlockSpec((bm, d_in), lambda i: (i, 0)),
          pl.BlockSpec((d_in, dh), lambda i: (0, 0)),
      ],
      out_specs=pl.BlockSpec((bm, dh), lambda i: (i, 0)),
      out_shape=jax.ShapeDtypeStruct((n, dh), jnp.float32),
  )(x, W1p)

  g1 = pl.pallas_call(
      _tc1_body,
      grid=(grid,),
      in_specs=[
          deg_spec,
          pl.BlockSpec((bm, dh), lambda i: (i, 0)),
      ],
      out_specs=pl.BlockSpec((bm, dh), lambda i: (i, 0)),
      out_shape=jax.ShapeDtypeStruct((n, dh), jnp.float32),
  )(deg2, h1)

  edge_agg = _edge_agg_kernel(n_pad, dh, e, chunk, fch)
  acc1 = edge_agg(g1, src, dst)[:, :n]

  g2 = pl.pallas_call(
      _tc2_body,
      grid=(grid,),
      in_specs=[
          acc_spec_h,
          pl.BlockSpec((bm, dh), lambda i: (i, 0)),
          deg_spec,
          pl.BlockSpec((1, dh), lambda i: (0, 0)),
          pl.BlockSpec((dh, d_out), lambda i: (0, 0)),
      ],
      out_specs=pl.BlockSpec((bm, d_out), lambda i: (i, 0)),
      out_shape=jax.ShapeDtypeStruct((n, d_out), jnp.float32),
  )(acc1, g1, deg2, b1p.reshape(1, dh), W2p)

  acc2 = edge_agg(g2, src, dst)[:, :n]

  out = pl.pallas_call(
      _tc3_body,
      grid=(grid,),
      in_specs=[
          acc_spec_o,
          pl.BlockSpec((bm, d_out), lambda i: (i, 0)),
          deg_spec,
          pl.BlockSpec((1, d_out), lambda i: (0, 0)),
      ],
      out_specs=pl.BlockSpec((bm, d_out), lambda i: (i, 0)),
      out_shape=jax.ShapeDtypeStruct((n, d_out), jnp.float32),
  )(acc2, g2, deg2, b2.reshape(1, d_out))

  return out


# speedup vs baseline: 33.0000x; 1.0252x over previous
import functools

import jax
import jax.numpy as jnp
from jax import lax
from jax.experimental import pallas as pl
from jax.experimental.pallas import tpu as pltpu
from jax.experimental.pallas import tpu_sc as plsc

NC = 2
NS = 16
NW = NC * NS


_D = 5


def _deg_kernel(n_pad, e, chunk):
  e_per_w = e // NW
  nch = e_per_w // chunk
  rows_per_tec = n_pad // NS

  mesh = plsc.VectorSubcoreMesh(
      core_axis_name="c", subcore_axis_name="s",
      num_cores=NC, num_subcores=NS)

  @functools.partial(
      pl.kernel,
      out_type=jax.ShapeDtypeStruct((NC * n_pad,), jnp.float32),
      mesh=mesh,
      scratch_types=(
          [pltpu.VMEM((chunk,), jnp.int32)] * _D +
          [pltpu.VMEM((((chunk + 15) // 16) * 16,), jnp.float32)] +
          [pltpu.VMEM((rows_per_tec,), jnp.float32)] +
          [pltpu.VMEM_SHARED((n_pad,), jnp.float32)] +
          [pltpu.SemaphoreType.DMA] * (2 * _D)
      ),
  )
  def k(dst_hbm, out_hbm, *scr):
    didx = scr[:_D]
    ones, fbuf, acc = scr[_D], scr[_D + 1], scr[_D + 2]
    sems = scr[_D + 3:]
    isem = sems[:_D]
    ssem = sems[_D:]
    core = lax.axis_index("c")
    sub = lax.axis_index("s")
    wid = core * NS + sub
    estart = wid * e_per_w

    def fill(i, _):
      ones[pl.ds(i * 16, 16)] = jnp.full((16,), 1.0, jnp.float32)
      return 0
    lax.fori_loop(0, (chunk + 15) // 16, fill, 0)

    def zero(i, _):
      fbuf[pl.ds(i * 16, 16)] = jnp.zeros((16,), jnp.float32)
      return 0
    lax.fori_loop(0, rows_per_tec // 16, zero, 0)
    pltpu.sync_copy(fbuf, acc.at[pl.ds(sub * rows_per_tec, rows_per_tec)])
    plsc.subcore_barrier()

    def idx_load(i, b):
      return pltpu.make_async_copy(
          dst_hbm.at[pl.ds(estart + i * chunk, chunk)], didx[b], isem[b])

    for b in range(_D):
      idx_load(b, b).start()

    def group(g, _):
      for b in range(_D):
        i = g * _D + b
        idx_load(i, b).wait()
        sc = pltpu.make_async_copy(
            ones.at[pl.ds(0, chunk)], acc.at[didx[b]], ssem[b])
        sc.start(add=True)
        sc.wait()

        @pl.when(i + _D < nch)
        def _():
          idx_load(i + _D, b).start()
      return 0
    lax.fori_loop(0, nch // _D, group, 0)
    plsc.subcore_barrier()

    pltpu.sync_copy(
        acc.at[pl.ds(sub * rows_per_tec, rows_per_tec)],
        out_hbm.at[pl.ds(core * n_pad + sub * rows_per_tec, rows_per_tec)])

  return k


def _edge_agg_kernel(n_pad, d, e, chunk, fch):
  e_per_w = e // NW
  nch = e_per_w // chunk
  rows_per_tec = n_pad // NS
  nf = rows_per_tec // fch

  mesh = plsc.VectorSubcoreMesh(
      core_axis_name="c", subcore_axis_name="s",
      num_cores=NC, num_subcores=NS)

  @functools.partial(
      pl.kernel,
      out_type=jax.ShapeDtypeStruct((NC, n_pad, d), jnp.float32),
      mesh=mesh,
      scratch_types=(
          [pltpu.VMEM((chunk,), jnp.int32)] * (2 * _D) +
          [pltpu.VMEM((chunk,), jnp.int32)] * (2 * _D) +
          [pltpu.VMEM((chunk, d), jnp.float32)] * _D +
          [pltpu.VMEM((fch, d), jnp.float32)] +
          [pltpu.VMEM_SHARED((n_pad, d), jnp.float32)] +
          [pltpu.SemaphoreType.DMA] * (6 * _D)
      ),
  )
  def k(g_hbm, src_hbm, dst_hbm, out_hbm, *scr):
    sidx = scr[:2 * _D]
    didx = scr[2 * _D:4 * _D]
    rows = scr[4 * _D:5 * _D]
    fbuf, acc = scr[5 * _D], scr[5 * _D + 1]
    sems = scr[5 * _D + 2:]
    isem_s = sems[:2 * _D]
    isem_d = sems[2 * _D:4 * _D]
    gsem = sems[4 * _D:5 * _D]
    ssem = sems[5 * _D:]
    core = lax.axis_index("c")
    sub = lax.axis_index("s")
    wid = core * NS + sub
    estart = wid * e_per_w

    def zero(i, _):
      r = i // (d // 16)
      c = (i % (d // 16)) * 16
      fbuf[r, pl.ds(c, 16)] = jnp.zeros((16,), jnp.float32)
      return 0
    lax.fori_loop(0, fch * (d // 16), zero, 0)

    def zero_dma(f):
      r0 = sub * rows_per_tec + f * fch
      return pltpu.make_async_copy(fbuf, acc.at[pl.ds(r0, fch)], ssem[f % _D])
    for f in range(nf):
      zero_dma(f).start()
    for f in range(nf):
      zero_dma(f).wait()
    plsc.subcore_barrier()

    def sidx_load(i, c):
      return pltpu.make_async_copy(
          src_hbm.at[pl.ds(estart + i * chunk, chunk)], sidx[c], isem_s[c])

    def didx_load(i, c):
      return pltpu.make_async_copy(
          dst_hbm.at[pl.ds(estart + i * chunk, chunk)], didx[c], isem_d[c])

    def gather(b, c):
      return pltpu.make_async_copy(g_hbm.at[sidx[c]], rows[b], gsem[b])

    def scatter(b, c):
      return pltpu.make_async_copy(rows[b], acc.at[didx[c]], ssem[b])

    for c in range(2 * _D):
      sidx_load(c, c).start()
      didx_load(c, c).start()
    for b in range(_D - 1):
      sidx_load(b, b).wait()
      gather(b, b).start()

    def group(g, _):
      for h in range(2):
        for b in range(_D):
          i = g * (2 * _D) + h * _D + b
          c = h * _D + b
          jb = (b + _D - 1) % _D
          jc = (c + _D - 1) % (2 * _D)
          pc = (c + 2 * _D - 1) % (2 * _D)

          @pl.when((i + _D - 1 < nch) & (i >= 1))
          def _():
            scatter(jb, pc).wait()

          @pl.when(i + _D - 1 < nch)
          def _():
            sidx_load(i + _D - 1, jc).wait()
            gather(jb, jc).start()

          gather(b, c).wait()
          didx_load(i, c).wait()
          scatter(b, c).start(add=True)

          @pl.when((i >= 1) & (i + 2 * _D - 1 < nch))
          def _():
            sidx_load(i + 2 * _D - 1, pc).start()
            didx_load(i + 2 * _D - 1, pc).start()
      return 0
    lax.fori_loop(0, nch // (2 * _D), group, 0)

    for b in range(_D):
      i = nch - _D + b
      scatter(i % _D, i % (2 * _D)).wait()
    plsc.subcore_barrier()

    def flush_dma(f):
      r0 = sub * rows_per_tec + f * fch
      return pltpu.make_async_copy(
          acc.at[pl.ds(r0, fch)], out_hbm.at[core, pl.ds(r0, fch)],
          ssem[f % _D])
    for f in range(nf):
      flush_dma(f).start()
    for f in range(nf):
      flush_dma(f).wait()

  return k


def _dis_from_deg(deg_ref):
  deg = deg_ref[:, 0:1] + deg_ref[:, 1:2] + 1.0
  return lax.rsqrt(jnp.maximum(deg, 1.0))


def _tc0_body(x_ref, w1_ref, h1_ref):
  h1_ref[...] = jnp.dot(x_ref[...], w1_ref[...],
                        preferred_element_type=jnp.float32,
                        precision=lax.Precision.HIGHEST)


def _tc1_body(deg_ref, h1_ref, g1_ref):
  dis = _dis_from_deg(deg_ref)
  g1_ref[...] = h1_ref[...] * dis


def _tc2_body(acc_ref, g1_ref, deg_ref, b1_ref, w2_ref, g2_ref):
  dis = _dis_from_deg(deg_ref)
  s = acc_ref[0] + acc_ref[1] + g1_ref[...]
  out1 = jnp.maximum(s * dis + b1_ref[...], 0.0)
  h2 = jnp.dot(out1, w2_ref[...], preferred_element_type=jnp.float32,
               precision=lax.Precision.HIGHEST)
  g2_ref[...] = h2 * dis


def _tc3_body(acc_ref, g2_ref, deg_ref, b2_ref, o_ref):
  dis = _dis_from_deg(deg_ref)
  s = acc_ref[0] + acc_ref[1] + g2_ref[...]
  o = s * dis + b2_ref[...]
  m = jnp.max(o, axis=1, keepdims=True)
  z = o - m
  lse = jnp.log(jnp.sum(jnp.exp(z), axis=1, keepdims=True))
  o_ref[...] = z - lse


@jax.jit
def kernel(x, edge_index, W1, b1, W2, b2):
  n, d_in = x.shape
  e = edge_index.shape[1]
  d_hid = W1.shape[1]
  d_out = W2.shape[1]

  src = edge_index[0]
  dst = edge_index[1]

  dh = 128
  W1p = jnp.pad(W1, ((0, 0), (0, dh - d_hid)))
  b1p = jnp.pad(b1, (0, dh - d_hid))
  W2p = jnp.pad(W2, ((0, dh - d_hid), (0, 0)))

  n_pad = ((n + (128 * NS) - 1) // (128 * NS)) * (128 * NS)
  chunk = 40
  fch = 128

  deg2 = _deg_kernel(n_pad, e, 2 * chunk)(dst)
  deg2 = deg2.reshape(NC, n_pad)[:, :n].T

  bm = 1000
  grid = n // bm

  deg_spec = pl.BlockSpec((bm, NC), lambda i: (i, 0))
  acc_spec_h = pl.BlockSpec((NC, bm, dh), lambda i: (0, i, 0))
  acc_spec_o = pl.BlockSpec((NC, bm, d_out), lambda i: (0, i, 0))

  h1 = pl.pallas_call(
      _tc0_body,
      grid=(grid,),
      in_specs=[
          pl.BlockSpec((bm, d_in), lambda i: (i, 0)),
          pl.BlockSpec((d_in, dh), lambda i: (0, 0)),
      ],
      out_specs=pl.BlockSpec((bm, dh), lambda i: (i, 0)),
      out_shape=jax.ShapeDtypeStruct((n, dh), jnp.float32),
  )(x, W1p)

  g1 = pl.pallas_call(
      _tc1_body,
      grid=(grid,),
      in_specs=[
          deg_spec,
          pl.BlockSpec((bm, dh), lambda i: (i, 0)),
      ],
      out_specs=pl.BlockSpec((bm, dh), lambda i: (i, 0)),
      out_shape=jax.ShapeDtypeStruct((n, dh), jnp.float32),
  )(deg2, h1)

  edge_agg = _edge_agg_kernel(n_pad, dh, e, chunk, fch)
  acc1 = edge_agg(g1, src, dst)[:, :n]

  g2 = pl.pallas_call(
      _tc2_body,
      grid=(grid,),
      in_specs=[
          acc_spec_h,
          pl.BlockSpec((bm, dh), lambda i: (i, 0)),
          deg_spec,
          pl.BlockSpec((1, dh), lambda i: (0, 0)),
          pl.BlockSpec((dh, d_out), lambda i: (0, 0)),
      ],
      out_specs=pl.BlockSpec((bm, d_out), lambda i: (i, 0)),
      out_shape=jax.ShapeDtypeStruct((n, d_out), jnp.float32),
  )(acc1, g1, deg2, b1p.reshape(1, dh), W2p)

  acc2 = edge_agg(g2, src, dst)[:, :n]

  out = pl.pallas_call(
      _tc3_body,
      grid=(grid,),
      in_specs=[
          acc_spec_o,
          pl.BlockSpec((bm, d_out), lambda i: (i, 0)),
          deg_spec,
          pl.BlockSpec((1, d_out), lambda i: (0, 0)),
      ],
      out_specs=pl.BlockSpec((bm, d_out), lambda i: (i, 0)),
      out_shape=jax.ShapeDtypeStruct((n, d_out), jnp.float32),
  )(acc2, g2, deg2, b2.reshape(1, d_out))

  return out
